# baseline (speedup 1.0000x reference)
import jax
import jax.numpy as jnp
from jax import lax
from jax.experimental import pallas as pl
from jax.experimental.pallas import tpu as pltpu
from jax.experimental.pallas import tpu_sc as plsc

HIDDEN_DIM = 128
MAX_DEPTH = 64
BATCH = 128
MAX_NODES = 256
VALUE_LEN = 32

NUM_CORES = 2
NUM_SUBCORES = 16
NUM_WORKERS = NUM_CORES * NUM_SUBCORES
LANES = 16

TOTAL_ROWS = BATCH * MAX_NODES
ROWS_PER_WORKER = TOTAL_ROWS // NUM_WORKERS
CHUNK = 4
NCH = ROWS_PER_WORKER // CHUNK
VCOLS = 128
SUBBLOCKS = ROWS_PER_WORKER * VALUE_LEN // VCOLS
SUPER = 32
NSUP = ROWS_PER_WORKER // SUPER
CH_PER_SUP = SUPER // CHUNK
NVSET = 4
NJ = HIDDEN_DIM // LANES


def _body(nt_hbm, dp_hbm, nv_hbm, node_tab, value_tab, depth_tab, out_hbm,
          nidx, didx, vidx, nb0, nb1, db0, db1, vb0, vb1, vb2, vb3, obuf,
          vsm0, vsm1, vsm2, vsm3, nsm):
    nbufs = (nb0, nb1)
    dbufs = (db0, db1)
    vbufs = (vb0, vb1, vb2, vb3)
    vsems = (vsm0, vsm1, vsm2, vsm3)

    wid = lax.axis_index("s") * NUM_CORES + lax.axis_index("c")

    pltpu.sync_copy(nt_hbm.at[pl.ds(wid * NSUP, NSUP)], nidx)
    pltpu.sync_copy(dp_hbm.at[pl.ds(wid * NSUP, NSUP)], didx)
    pltpu.sync_copy(nv_hbm.at[pl.ds(wid * SUBBLOCKS, SUBBLOCKS)], vidx)

    dbase = wid * MAX_DEPTH

    def clamp_body(i, _):
        for half in range(SUPER // LANES):
            sl = pl.ds(half * LANES, LANES)
            didx[i, sl] = jnp.clip(didx[i, sl], 0, MAX_DEPTH - 1) + dbase
        return 0
    lax.fori_loop(0, NSUP, clamp_body, 0)

    def fire_nd(s, g):
        return [pltpu.async_copy(node_tab.at[nidx.at[s]], nbufs[g], nsm),
                pltpu.async_copy(depth_tab.at[didx.at[s]], dbufs[g], nsm)]

    def fire_value(c, p):
        return pltpu.async_copy(value_tab.at[vidx.at[c]], vbufs[p], vsems[p])

    scale = jnp.float32(1.0 / VALUE_LEN)

    def compute_store(cc, p, g):
        vb = vbufs[p]
        nb = nbufs[g]
        db = dbufs[g]

        def node_body(n, _):
            rowb = n * VALUE_LEN

            def l_body(l2, accs):
                l0 = 4 * l2
                hi_mask = jnp.uint32(0xFFFF0000)
                for u in range(4):
                    new = []
                    for k in range(NJ // 2):
                        wi = vb[rowb + l0 + u, pl.ds(k * LANES, LANES)]
                        a = lax.bitcast_convert_type(wi << 16, jnp.float32)
                        b = lax.bitcast_convert_type(wi & hi_mask,
                                                     jnp.float32)
                        new.append(accs[2 * k] + a)
                        new.append(accs[2 * k + 1] + b)
                    accs = tuple(new)
                return accs

            accs = tuple(jnp.zeros((LANES,), jnp.float32) for _ in range(NJ))
            accs = lax.fori_loop(0, VALUE_LEN // 4, l_body, accs)

            col = CHUNK * cc + n
            for j in range(NJ):
                obuf[col, pl.ds(j * LANES, LANES)] = (
                    accs[j] * scale
                    + nb[col, pl.ds(j * LANES, LANES)]
                    + db[col, pl.ds(j * LANES, LANES)])
            return 0

        lax.fori_loop(0, CHUNK, node_body, 0)

    for d in fire_nd(0, 0):
        d.wait()
    fire_value(0, 0).wait()
    fire_value(1, 1).wait()

    def sup_pair_body(sp, _):
        for gg in range(2):
            s = 2 * sp + gg
            nds = fire_nd(jnp.minimum(s + 1, NSUP - 1), 1 - gg)
            for pp in range(CH_PER_SUP // 2):
                c0 = CH_PER_SUP * s + 2 * pp
                vd0 = fire_value(jnp.minimum(c0 + 2, NCH - 1),
                                 (2 * pp + 2) % NVSET)
                vd1 = fire_value(jnp.minimum(c0 + 3, NCH - 1),
                                 (2 * pp + 3) % NVSET)
                compute_store(2 * pp, (2 * pp) % NVSET, gg)
                compute_store(2 * pp + 1, (2 * pp + 1) % NVSET, gg)
                vd0.wait()
                vd1.wait()
            pltpu.sync_copy(
                obuf,
                out_hbm.at[pl.ds(wid * ROWS_PER_WORKER + s * SUPER, SUPER)])
            for d in nds:
                d.wait()
        return 0

    lax.fori_loop(0, NSUP // 2, sup_pair_body, 0)


@jax.jit
def _tree_embedding(nt2, dp2, nv2, node_table, value_table, depth_table):
    mesh = plsc.VectorSubcoreMesh(core_axis_name="c", subcore_axis_name="s")
    return pl.kernel(
        _body,
        out_type=jax.ShapeDtypeStruct((TOTAL_ROWS, HIDDEN_DIM), jnp.float32),
        mesh=mesh,
        compiler_params=pltpu.CompilerParams(use_tc_tiling_on_sc=False),
        scratch_types=[
            pltpu.VMEM((NSUP, SUPER), jnp.int32),
            pltpu.VMEM((NSUP, SUPER), jnp.int32),
            pltpu.VMEM((SUBBLOCKS, VCOLS), jnp.int32),
            pltpu.VMEM((SUPER, HIDDEN_DIM), jnp.float32),
            pltpu.VMEM((SUPER, HIDDEN_DIM), jnp.float32),
            pltpu.VMEM((SUPER, HIDDEN_DIM), jnp.float32),
            pltpu.VMEM((SUPER, HIDDEN_DIM), jnp.float32),
            pltpu.VMEM((CHUNK * VALUE_LEN, HIDDEN_DIM // 2), jnp.uint32),
            pltpu.VMEM((CHUNK * VALUE_LEN, HIDDEN_DIM // 2), jnp.uint32),
            pltpu.VMEM((CHUNK * VALUE_LEN, HIDDEN_DIM // 2), jnp.uint32),
            pltpu.VMEM((CHUNK * VALUE_LEN, HIDDEN_DIM // 2), jnp.uint32),
            pltpu.VMEM((SUPER, HIDDEN_DIM), jnp.float32),
            pltpu.SemaphoreType.DMA,
            pltpu.SemaphoreType.DMA,
            pltpu.SemaphoreType.DMA,
            pltpu.SemaphoreType.DMA,
            pltpu.SemaphoreType.DMA,
        ],
    )(nt2, dp2, nv2, node_table, value_table, depth_table)


def kernel(node_types, node_values, depth, node_table, value_table, depth_table):
    nt2 = node_types.reshape(TOTAL_ROWS // SUPER, SUPER).astype(jnp.int32)
    dp2 = depth.reshape(TOTAL_ROWS // SUPER, SUPER).astype(jnp.int32)
    nv2 = node_values.reshape(TOTAL_ROWS * VALUE_LEN // VCOLS,
                              VCOLS).astype(jnp.int32)
    dtab_rep = jnp.tile(depth_table, (NUM_WORKERS, 1))
    blk = jnp.arange(LANES)
    p_block = jnp.stack([blk, blk + LANES], axis=1).reshape(2 * LANES)
    perm = (jnp.arange(0, HIDDEN_DIM, 2 * LANES)[:, None]
            + p_block[None, :]).reshape(-1)
    vtab_bf = value_table[:, perm].astype(jnp.bfloat16)
    vtab_u32 = lax.bitcast_convert_type(
        vtab_bf.reshape(-1, HIDDEN_DIM // 2, 2), jnp.uint32)
    out = _tree_embedding(nt2, dp2, nv2, node_table, vtab_u32, dtab_rep)
    return out.reshape(BATCH, MAX_NODES, HIDDEN_DIM)


# speedup vs baseline: 2.1841x; 2.1841x over previous
import jax
import jax.numpy as jnp
from jax import lax
from jax.experimental import pallas as pl
from jax.experimental.pallas import tpu as pltpu
from jax.experimental.pallas import tpu_sc as plsc

HIDDEN_DIM = 128
MAX_DEPTH = 64
BATCH = 128
MAX_NODES = 256
VALUE_LEN = 32

NUM_CORES = 2
NUM_SUBCORES = 16
NUM_WORKERS = NUM_CORES * NUM_SUBCORES
LANES = 16

TOTAL_ROWS = BATCH * MAX_NODES
ROWS_PER_WORKER = TOTAL_ROWS // NUM_WORKERS
CHUNK = 4
NCH = ROWS_PER_WORKER // CHUNK
VCOLS = 128
SUBBLOCKS = ROWS_PER_WORKER * VALUE_LEN // VCOLS
SUPER = 32
NSUP = ROWS_PER_WORKER // SUPER
CH_PER_SUP = SUPER // CHUNK
NVSET = 4
NJ = HIDDEN_DIM // LANES


def _body(nt_hbm, dp_hbm, nv_hbm, node_tab, value_tab, depth_tab, out_hbm,
          nidx, didx, vidx, nb0, nb1, db0, db1, vb0, vb1, vb2, vb3, obuf,
          vsm0, vsm1, vsm2, vsm3, nsm):
    nbufs = (nb0, nb1)
    dbufs = (db0, db1)
    vbufs = (vb0, vb1, vb2, vb3)
    vsems = (vsm0, vsm1, vsm2, vsm3)

    wid = lax.axis_index("s") * NUM_CORES + lax.axis_index("c")

    pltpu.sync_copy(nt_hbm.at[pl.ds(wid * NSUP, NSUP)], nidx)
    pltpu.sync_copy(dp_hbm.at[pl.ds(wid * NSUP, NSUP)], didx)
    pltpu.sync_copy(nv_hbm.at[pl.ds(wid * SUBBLOCKS, SUBBLOCKS)], vidx)

    dbase = wid * MAX_DEPTH

    def clamp_body(i, _):
        for half in range(SUPER // LANES):
            sl = pl.ds(half * LANES, LANES)
            didx[i, sl] = jnp.clip(didx[i, sl], 0, MAX_DEPTH - 1) + dbase
        return 0
    lax.fori_loop(0, NSUP, clamp_body, 0)

    def fire_nd(s, g):
        return [pltpu.async_copy(node_tab.at[nidx.at[s]], nbufs[g], nsm),
                pltpu.async_copy(depth_tab.at[didx.at[s]], dbufs[g], nsm)]

    def fire_value(c, p):
        return pltpu.async_copy(value_tab.at[vidx.at[c]], vbufs[p], vsems[p])

    scale = jnp.float32(1.0 / VALUE_LEN)

    def compute_store(cc, p, g):
        vb = vbufs[p]
        nb = nbufs[g]
        db = dbufs[g]

        def node_body(n, _):
            rowb = n * VALUE_LEN

            def l_body(l2, accs):
                l0 = 4 * l2
                for u in range(4):
                    accs = tuple(
                        accs[j] + vb[rowb + l0 + u, pl.ds(j * LANES, LANES)]
                        for j in range(NJ))
                return accs

            accs = tuple(jnp.zeros((LANES,), jnp.float32) for _ in range(NJ))
            accs = lax.fori_loop(0, VALUE_LEN // 4, l_body, accs)

            col = CHUNK * cc + n
            for j in range(NJ):
                obuf[col, pl.ds(j * LANES, LANES)] = (
                    accs[j] * scale
                    + nb[col, pl.ds(j * LANES, LANES)]
                    + db[col, pl.ds(j * LANES, LANES)])
            return 0

        lax.fori_loop(0, CHUNK, node_body, 0)

    for d in fire_nd(0, 0):
        d.wait()
    fire_value(0, 0).wait()
    fire_value(1, 1).wait()

    def sup_pair_body(sp, _):
        for gg in range(2):
            s = 2 * sp + gg
            nds = fire_nd(jnp.minimum(s + 1, NSUP - 1), 1 - gg)
            for pp in range(CH_PER_SUP // 2):
                c0 = CH_PER_SUP * s + 2 * pp
                vd0 = fire_value(jnp.minimum(c0 + 2, NCH - 1),
                                 (2 * pp + 2) % NVSET)
                vd1 = fire_value(jnp.minimum(c0 + 3, NCH - 1),
                                 (2 * pp + 3) % NVSET)
                compute_store(2 * pp, (2 * pp) % NVSET, gg)
                compute_store(2 * pp + 1, (2 * pp + 1) % NVSET, gg)
                vd0.wait()
                vd1.wait()
            pltpu.sync_copy(
                obuf,
                out_hbm.at[pl.ds(wid * ROWS_PER_WORKER + s * SUPER, SUPER)])
            for d in nds:
                d.wait()
        return 0

    lax.fori_loop(0, NSUP // 2, sup_pair_body, 0)


@jax.jit
def _tree_embedding(nt2, dp2, nv2, node_table, value_table, depth_table):
    mesh = plsc.VectorSubcoreMesh(core_axis_name="c", subcore_axis_name="s")
    return pl.kernel(
        _body,
        out_type=jax.ShapeDtypeStruct((TOTAL_ROWS, HIDDEN_DIM), jnp.float32),
        mesh=mesh,
        scratch_types=[
            pltpu.VMEM((NSUP, SUPER), jnp.int32),
            pltpu.VMEM((NSUP, SUPER), jnp.int32),
            pltpu.VMEM((SUBBLOCKS, VCOLS), jnp.int32),
            pltpu.VMEM((SUPER, HIDDEN_DIM), jnp.float32),
            pltpu.VMEM((SUPER, HIDDEN_DIM), jnp.float32),
            pltpu.VMEM((SUPER, HIDDEN_DIM), jnp.float32),
            pltpu.VMEM((SUPER, HIDDEN_DIM), jnp.float32),
            pltpu.VMEM((CHUNK * VALUE_LEN, HIDDEN_DIM), jnp.float32),
            pltpu.VMEM((CHUNK * VALUE_LEN, HIDDEN_DIM), jnp.float32),
            pltpu.VMEM((CHUNK * VALUE_LEN, HIDDEN_DIM), jnp.float32),
            pltpu.VMEM((CHUNK * VALUE_LEN, HIDDEN_DIM), jnp.float32),
            pltpu.VMEM((SUPER, HIDDEN_DIM), jnp.float32),
            pltpu.SemaphoreType.DMA,
            pltpu.SemaphoreType.DMA,
            pltpu.SemaphoreType.DMA,
            pltpu.SemaphoreType.DMA,
            pltpu.SemaphoreType.DMA,
        ],
    )(nt2, dp2, nv2, node_table, value_table, depth_table)


def kernel(node_types, node_values, depth, node_table, value_table, depth_table):
    nt2 = node_types.reshape(TOTAL_ROWS // SUPER, SUPER).astype(jnp.int32)
    dp2 = depth.reshape(TOTAL_ROWS // SUPER, SUPER).astype(jnp.int32)
    nv2 = node_values.reshape(TOTAL_ROWS * VALUE_LEN // VCOLS,
                              VCOLS).astype(jnp.int32)
    dtab_rep = jnp.tile(depth_table, (NUM_WORKERS, 1))
    out = _tree_embedding(nt2, dp2, nv2, node_table, value_table, dtab_rep)
    return out.reshape(BATCH, MAX_NODES, HIDDEN_DIM)


# speedup vs baseline: 2.2192x; 1.0161x over previous
import jax
import jax.numpy as jnp
from jax import lax
from jax.experimental import pallas as pl
from jax.experimental.pallas import tpu as pltpu
from jax.experimental.pallas import tpu_sc as plsc

HIDDEN_DIM = 128
MAX_DEPTH = 64
BATCH = 128
MAX_NODES = 256
VALUE_LEN = 32

NUM_CORES = 2
NUM_SUBCORES = 16
NUM_WORKERS = NUM_CORES * NUM_SUBCORES
LANES = 16

TOTAL_ROWS = BATCH * MAX_NODES
ROWS_PER_WORKER = TOTAL_ROWS // NUM_WORKERS
CHUNK = 4
NCH = ROWS_PER_WORKER // CHUNK
VCOLS = 128
SUBBLOCKS = ROWS_PER_WORKER * VALUE_LEN // VCOLS
SUPER = 32
NSUP = ROWS_PER_WORKER // SUPER
CH_PER_SUP = SUPER // CHUNK
NVSET = 4
NJ = HIDDEN_DIM // LANES


def _body(nt_hbm, dp_hbm, nv_hbm, node_tab, value_tab, depth_tab, out_hbm,
          nidx, didx, vidx, nb0, nb1, db0, db1, vb0, vb1, vb2, vb3, ob0, ob1,
          vsm0, vsm1, vsm2, vsm3, nsm, osm0, osm1):
    nbufs = (nb0, nb1)
    dbufs = (db0, db1)
    vbufs = (vb0, vb1, vb2, vb3)
    obufs = (ob0, ob1)
    vsems = (vsm0, vsm1, vsm2, vsm3)
    osems = (osm0, osm1)

    wid = lax.axis_index("s") * NUM_CORES + lax.axis_index("c")

    pltpu.sync_copy(nt_hbm.at[pl.ds(wid * NSUP, NSUP)], nidx)
    pltpu.sync_copy(dp_hbm.at[pl.ds(wid * NSUP, NSUP)], didx)
    pltpu.sync_copy(nv_hbm.at[pl.ds(wid * SUBBLOCKS, SUBBLOCKS)], vidx)

    dbase = wid * MAX_DEPTH

    def clamp_body(i, _):
        for half in range(SUPER // LANES):
            sl = pl.ds(half * LANES, LANES)
            didx[i, sl] = jnp.clip(didx[i, sl], 0, MAX_DEPTH - 1) + dbase
        return 0
    lax.fori_loop(0, NSUP, clamp_body, 0)

    def fire_nd(s, g):
        return [pltpu.async_copy(node_tab.at[nidx.at[s]], nbufs[g], nsm),
                pltpu.async_copy(depth_tab.at[didx.at[s]], dbufs[g], nsm)]

    def fire_value(c, p):
        return pltpu.async_copy(value_tab.at[vidx.at[c]], vbufs[p], vsems[p])

    scale = jnp.float32(1.0 / VALUE_LEN)

    def compute_store(cc, p, g):
        vb = vbufs[p]
        nb = nbufs[g]
        db = dbufs[g]
        obuf = obufs[g]

        def node_body(n, _):
            rowb = n * VALUE_LEN

            def l_body(l2, accs):
                l0 = 4 * l2
                for u in range(4):
                    accs = tuple(
                        accs[j] + vb[rowb + l0 + u, pl.ds(j * LANES, LANES)]
                        for j in range(NJ))
                return accs

            accs = tuple(jnp.zeros((LANES,), jnp.float32) for _ in range(NJ))
            accs = lax.fori_loop(0, VALUE_LEN // 4, l_body, accs)

            col = CHUNK * cc + n
            for j in range(NJ):
                obuf[col, pl.ds(j * LANES, LANES)] = (
                    accs[j] * scale
                    + nb[col, pl.ds(j * LANES, LANES)]
                    + db[col, pl.ds(j * LANES, LANES)])
            return 0

        lax.fori_loop(0, CHUNK, node_body, 0)

    for d in fire_nd(0, 0):
        d.wait()
    fire_value(0, 0).wait()
    fire_value(1, 1).wait()

    def sup_pair_body(sp, _):
        for gg in range(2):
            s = 2 * sp + gg
            nds = fire_nd(jnp.minimum(s + 1, NSUP - 1), 1 - gg)

            @pl.when(s >= 2)
            def _():
                pltpu.make_async_copy(
                    obufs[gg],
                    out_hbm.at[pl.ds(wid * ROWS_PER_WORKER
                                     + (s - 2) * SUPER, SUPER)],
                    osems[gg]).wait()

            for pp in range(CH_PER_SUP // 2):
                c0 = CH_PER_SUP * s + 2 * pp
                vd0 = fire_value(jnp.minimum(c0 + 2, NCH - 1),
                                 (2 * pp + 2) % NVSET)
                vd1 = fire_value(jnp.minimum(c0 + 3, NCH - 1),
                                 (2 * pp + 3) % NVSET)
                compute_store(2 * pp, (2 * pp) % NVSET, gg)
                compute_store(2 * pp + 1, (2 * pp + 1) % NVSET, gg)
                vd0.wait()
                vd1.wait()
            pltpu.async_copy(
                obufs[gg],
                out_hbm.at[pl.ds(wid * ROWS_PER_WORKER + s * SUPER, SUPER)],
                osems[gg])
            for d in nds:
                d.wait()
        return 0

    lax.fori_loop(0, NSUP // 2, sup_pair_body, 0)

    for gg in range(2):
        s_last = NSUP - 2 + gg
        pltpu.make_async_copy(
            obufs[gg],
            out_hbm.at[pl.ds(wid * ROWS_PER_WORKER + s_last * SUPER, SUPER)],
            osems[gg]).wait()


@jax.jit
def _tree_embedding(nt2, dp2, nv2, node_table, value_table, depth_table):
    mesh = plsc.VectorSubcoreMesh(core_axis_name="c", subcore_axis_name="s")
    return pl.kernel(
        _body,
        out_type=jax.ShapeDtypeStruct((TOTAL_ROWS, HIDDEN_DIM), jnp.float32),
        mesh=mesh,
        scratch_types=[
            pltpu.VMEM((NSUP, SUPER), jnp.int32),
            pltpu.VMEM((NSUP, SUPER), jnp.int32),
            pltpu.VMEM((SUBBLOCKS, VCOLS), jnp.int32),
            pltpu.VMEM((SUPER, HIDDEN_DIM), jnp.float32),
            pltpu.VMEM((SUPER, HIDDEN_DIM), jnp.float32),
            pltpu.VMEM((SUPER, HIDDEN_DIM), jnp.float32),
            pltpu.VMEM((SUPER, HIDDEN_DIM), jnp.float32),
            pltpu.VMEM((CHUNK * VALUE_LEN, HIDDEN_DIM), jnp.float32),
            pltpu.VMEM((CHUNK * VALUE_LEN, HIDDEN_DIM), jnp.float32),
            pltpu.VMEM((CHUNK * VALUE_LEN, HIDDEN_DIM), jnp.float32),
            pltpu.VMEM((CHUNK * VALUE_LEN, HIDDEN_DIM), jnp.float32),
            pltpu.VMEM((SUPER, HIDDEN_DIM), jnp.float32),
            pltpu.VMEM((SUPER, HIDDEN_DIM), jnp.float32),
            pltpu.SemaphoreType.DMA,
            pltpu.SemaphoreType.DMA,
            pltpu.SemaphoreType.DMA,
            pltpu.SemaphoreType.DMA,
            pltpu.SemaphoreType.DMA,
            pltpu.SemaphoreType.DMA,
            pltpu.SemaphoreType.DMA,
        ],
    )(nt2, dp2, nv2, node_table, value_table, depth_table)


def kernel(node_types, node_values, depth, node_table, value_table, depth_table):
    nt2 = node_types.reshape(TOTAL_ROWS // SUPER, SUPER).astype(jnp.int32)
    dp2 = depth.reshape(TOTAL_ROWS // SUPER, SUPER).astype(jnp.int32)
    nv2 = node_values.reshape(TOTAL_ROWS * VALUE_LEN // VCOLS,
                              VCOLS).astype(jnp.int32)
    dtab_rep = jnp.tile(depth_table, (NUM_WORKERS, 1))
    out = _tree_embedding(nt2, dp2, nv2, node_table, value_table, dtab_rep)
    return out.reshape(BATCH, MAX_NODES, HIDDEN_DIM)
